# baseline (speedup 1.0000x reference)
import functools

import jax
import jax.numpy as jnp
from jax import lax
from jax.experimental import pallas as pl
from jax.experimental.pallas import tpu as pltpu
from jax.experimental.pallas import tpu_sc as plsc

V = 1000000
B = 4096
L = 200
D = 64
H = 32
NC = 2
NS = 16
NW = NC * NS
BPW = B // NW
LA = 128
LB = L - LA
NV = D // 16
TB = 8192

PERM = (
    [2 * p for p in range(16)]
    + [2 * p + 1 for p in range(16)]
    + [32 + 2 * p for p in range(16)]
    + [33 + 2 * p for p in range(16)]
)


def _relayout_tc(emb_t):
    def body(in_ref, o_ref):
        t = in_ref[...].T.astype(jnp.bfloat16)
        o_ref[...] = jnp.concatenate([t, t], axis=1)

    return pl.pallas_call(
        body,
        grid=((V + TB - 1) // TB,),
        in_specs=[pl.BlockSpec((D, TB), lambda i: (0, i))],
        out_specs=pl.BlockSpec((TB, 2 * D), lambda i: (i, 0)),
        out_shape=jax.ShapeDtypeStruct((V, 2 * D), jnp.bfloat16),
    )(emb_t)


def _pool_sc(x2, table):
    mesh = plsc.VectorSubcoreMesh(core_axis_name="core", subcore_axis_name="subcore")

    @functools.partial(
        pl.kernel,
        out_type=jax.ShapeDtypeStruct((B, D), jnp.float32),
        mesh=mesh,
        scratch_types=[
            pltpu.VMEM((BPW, L), jnp.int32),
            pltpu.VMEM((L, D), jnp.bfloat16),
            pltpu.VMEM((L, D), jnp.bfloat16),
            pltpu.VMEM((BPW, D), jnp.float32),
            pltpu.SemaphoreType.DMA,
            pltpu.SemaphoreType.DMA,
            pltpu.SemaphoreType.DMA,
            pltpu.SemaphoreType.DMA,
        ],
        compiler_params=pltpu.CompilerParams(
            use_tc_tiling_on_sc=False, needs_layout_passes=False
        ),
    )
    def pool(x_hbm, tab_hbm, out_hbm, idx_v, rows0, rows1, out_v, sa0, sb0, sa1, sb1):
        wid = lax.axis_index("subcore") * NC + lax.axis_index("core")
        base = wid * BPW
        pltpu.sync_copy(x_hbm.at[pl.ds(base, BPW)], idx_v)

        def issue(b, rows, sa, sb):
            pltpu.async_copy(
                tab_hbm.at[idx_v.at[b, pl.ds(0, LA)]], rows.at[pl.ds(0, LA)], sa
            )
            pltpu.async_copy(
                tab_hbm.at[idx_v.at[b, pl.ds(LA, LB)]], rows.at[pl.ds(LA, LB)], sb
            )

        def wait(rows, sa, sb):
            pltpu.make_async_copy(
                tab_hbm.at[idx_v.at[0, pl.ds(0, LA)]], rows.at[pl.ds(0, LA)], sa
            ).wait()
            pltpu.make_async_copy(
                tab_hbm.at[idx_v.at[0, pl.ds(LA, LB)]], rows.at[pl.ds(LA, LB)], sb
            ).wait()

        def accum(b, rows):
            def body(r, accs):
                c0 = rows[r, pl.ds(0, 32)]
                c1 = rows[r, pl.ds(32, 32)]
                a0, a1 = plsc.unpack(c0, format=plsc.PackFormat.INTERLEAVED)
                a2, a3 = plsc.unpack(c1, format=plsc.PackFormat.INTERLEAVED)
                return (accs[0] + a0, accs[1] + a1, accs[2] + a2, accs[3] + a3)

            accs = lax.fori_loop(
                0, L, body, tuple(jnp.zeros((16,), jnp.float32) for _ in range(NV))
            )
            for i in range(NV):
                out_v[b, pl.ds(16 * i, 16)] = accs[i]

        issue(0, rows0, sa0, sb0)

        @pl.loop(0, BPW, step=2)
        def _(b):
            issue(b + 1, rows1, sa1, sb1)
            wait(rows0, sa0, sb0)
            accum(b, rows0)

            @pl.when(b + 2 < BPW)
            def _():
                issue(b + 2, rows0, sa0, sb0)

            wait(rows1, sa1, sb1)
            accum(b + 1, rows1)

        pltpu.sync_copy(out_v, out_hbm.at[pl.ds(base, BPW)])

    return pool(x2, table)


def _mlp_tc(pooled_sum, w1t, b1, w2, b2):
    def body(p_ref, w1_ref, b1_ref, w2_ref, b2_ref, o_ref):
        p = p_ref[...] * (1.0 / L)
        h = jnp.dot(p, w1_ref[...], preferred_element_type=jnp.float32) + b1_ref[...]
        h = jnp.maximum(h, 0.0)
        o_ref[...] = jnp.sum(h * w2_ref[...], axis=1, keepdims=True) + b2_ref[...]

    return pl.pallas_call(
        body,
        out_shape=jax.ShapeDtypeStruct((B, 1), jnp.float32),
    )(pooled_sum, w1t, b1, w2, b2)


def kernel(x, emb, W1, b1, W2, b2):
    table = _relayout_tc(emb.T).reshape(2 * V, D)
    pooled_sum = _pool_sc(x + x, table)
    w1t_perm = W1.T[jnp.array(PERM, dtype=jnp.int32), :]
    out = _mlp_tc(
        pooled_sum,
        w1t_perm,
        b1.reshape(1, H),
        W2.reshape(1, H),
        b2.reshape(1, 1),
    )
    return out.reshape(B)


# speedup vs baseline: 2.9036x; 2.9036x over previous
import functools

import jax
import jax.numpy as jnp
from jax import lax
from jax.experimental import pallas as pl
from jax.experimental.pallas import tpu as pltpu
from jax.experimental.pallas import tpu_sc as plsc

V = 1000000
B = 4096
L = 200
D = 64
H = 32
NC = 2
NS = 16
NW = NC * NS
BPW = B // NW
NACC = 5
NV = D // 16
TB = 16384


def _relayout_tc(emb_t):
    def body(in_ref, o_ref):
        t = in_ref[...].T
        o_ref[...] = jnp.concatenate([t, t], axis=1)

    return pl.pallas_call(
        body,
        grid=((V + TB - 1) // TB,),
        in_specs=[pl.BlockSpec((D, TB), lambda i: (0, i))],
        out_specs=pl.BlockSpec((TB, 2 * D), lambda i: (i, 0)),
        out_shape=jax.ShapeDtypeStruct((V, 2 * D), jnp.float32),
    )(emb_t)


def _pool_sc(x, table):
    mesh = plsc.VectorSubcoreMesh(core_axis_name="core", subcore_axis_name="subcore")

    @functools.partial(
        pl.kernel,
        out_type=jax.ShapeDtypeStruct((B, D), jnp.float32),
        mesh=mesh,
        scratch_types=[
            pltpu.VMEM((BPW, L), jnp.int32),
            pltpu.VMEM((L, BPW), jnp.int32),
            pltpu.VMEM((NACC, BPW, D), jnp.float32),
            pltpu.VMEM((BPW, D), jnp.float32),
        ]
        + [pltpu.SemaphoreType.DMA] * NACC,
        compiler_params=pltpu.CompilerParams(
            use_tc_tiling_on_sc=False, needs_layout_passes=False
        ),
    )
    def pool(x_hbm, table_hbm, out_hbm, idx_raw, idx_v, accs_v, out_v, *sems):
        wid = lax.axis_index("subcore") * NC + lax.axis_index("core")
        base = wid * BPW
        pltpu.sync_copy(x_hbm.at[pl.ds(base, BPW)], idx_raw)

        lanes = lax.iota(jnp.int32, 16)

        @pl.loop(0, L)
        def _(j):
            cols = jnp.zeros((16,), jnp.int32) + j
            for g in range(BPW // 16):
                v = plsc.load_gather(idx_raw, [lanes + 16 * g, cols])
                idx_v[j, pl.ds(16 * g, 16)] = v + v

        for k in range(NACC):
            pltpu.async_copy(table_hbm.at[idx_v.at[k]], accs_v.at[k], sems[k])

        @pl.loop(NACC, L, step=NACC)
        def _(j):
            for k in range(NACC):
                pltpu.make_async_copy(
                    table_hbm.at[idx_v.at[0]], accs_v.at[k], sems[k]
                ).wait()
                pltpu.async_copy(
                    table_hbm.at[idx_v.at[j + k]], accs_v.at[k], sems[k], add=True
                )

        for k in range(NACC):
            pltpu.make_async_copy(
                table_hbm.at[idx_v.at[0]], accs_v.at[k], sems[k]
            ).wait()

        @pl.loop(0, BPW)
        def _(b):
            for i in range(NV):
                s = pl.ds(16 * i, 16)
                acc = accs_v[0, b, s] + accs_v[1, b, s]
                for k in range(2, NACC):
                    acc = acc + accs_v[k, b, s]
                out_v[b, s] = acc

        pltpu.sync_copy(out_v, out_hbm.at[pl.ds(base, BPW)])

    return pool(x, table)


def _mlp_tc(pooled_sum, w1t, b1, w2, b2):
    def body(p_ref, w1_ref, b1_ref, w2_ref, b2_ref, o_ref):
        p = p_ref[...] * (1.0 / L)
        h = jnp.dot(p, w1_ref[...], preferred_element_type=jnp.float32) + b1_ref[...]
        h = jnp.maximum(h, 0.0)
        o_ref[...] = jnp.sum(h * w2_ref[...], axis=1, keepdims=True) + b2_ref[...]

    return pl.pallas_call(
        body,
        out_shape=jax.ShapeDtypeStruct((B, 1), jnp.float32),
    )(pooled_sum, w1t, b1, w2, b2)


def kernel(x, emb, W1, b1, W2, b2):
    table = _relayout_tc(emb.T).reshape(2 * V, D)
    pooled_sum = _pool_sc(x, table)
    out = _mlp_tc(
        pooled_sum,
        W1.T,
        b1.reshape(1, H),
        W2.reshape(1, H),
        b2.reshape(1, 1),
    )
    return out.reshape(B)


# speedup vs baseline: 2.9718x; 1.0235x over previous
import functools

import jax
import jax.numpy as jnp
from jax import lax
from jax.experimental import pallas as pl
from jax.experimental.pallas import tpu as pltpu
from jax.experimental.pallas import tpu_sc as plsc

V = 1000000
B = 4096
L = 200
D = 64
H = 32
NC = 2
NS = 16
NW = NC * NS
BPW = B // NW
NACC = 8
NV = D // 16
TB = 16384


def _relayout_tc(emb_t):
    def body(in_ref, o_ref):
        t = in_ref[...].T
        o_ref[...] = jnp.concatenate([t, t], axis=1)

    return pl.pallas_call(
        body,
        grid=((V + TB - 1) // TB,),
        in_specs=[pl.BlockSpec((D, TB), lambda i: (0, i))],
        out_specs=pl.BlockSpec((TB, 2 * D), lambda i: (i, 0)),
        out_shape=jax.ShapeDtypeStruct((V, 2 * D), jnp.float32),
    )(emb_t)


def _pool_sc(x, table):
    mesh = plsc.VectorSubcoreMesh(core_axis_name="core", subcore_axis_name="subcore")

    @functools.partial(
        pl.kernel,
        out_type=jax.ShapeDtypeStruct((B, D), jnp.float32),
        mesh=mesh,
        scratch_types=[
            pltpu.VMEM((BPW, L), jnp.int32),
            pltpu.VMEM((L, BPW), jnp.int32),
            pltpu.VMEM((NACC, BPW, D), jnp.float32),
            pltpu.VMEM((BPW, D), jnp.float32),
        ]
        + [pltpu.SemaphoreType.DMA] * NACC,
        compiler_params=pltpu.CompilerParams(
            use_tc_tiling_on_sc=False, needs_layout_passes=False
        ),
    )
    def pool(x_hbm, table_hbm, out_hbm, idx_raw, idx_v, accs_v, out_v, *sems):
        wid = lax.axis_index("subcore") * NC + lax.axis_index("core")
        base = wid * BPW
        pltpu.sync_copy(x_hbm.at[pl.ds(base, BPW)], idx_raw)

        lanes = lax.iota(jnp.int32, 16)

        @pl.loop(0, L)
        def _(j):
            cols = jnp.zeros((16,), jnp.int32) + j
            for g in range(BPW // 16):
                v = plsc.load_gather(idx_raw, [lanes + 16 * g, cols])
                idx_v[j, pl.ds(16 * g, 16)] = v + v

        for k in range(NACC):
            pltpu.async_copy(table_hbm.at[idx_v.at[k]], accs_v.at[k], sems[k])

        @pl.loop(NACC, L, step=NACC)
        def _(j):
            for k in range(NACC):
                pltpu.make_async_copy(
                    table_hbm.at[idx_v.at[0]], accs_v.at[k], sems[k]
                ).wait()
                pltpu.async_copy(
                    table_hbm.at[idx_v.at[j + k]], accs_v.at[k], sems[k], add=True
                )

        for k in range(NACC):
            pltpu.make_async_copy(
                table_hbm.at[idx_v.at[0]], accs_v.at[k], sems[k]
            ).wait()

        @pl.loop(0, BPW)
        def _(b):
            for i in range(NV):
                s = pl.ds(16 * i, 16)
                acc = accs_v[0, b, s] + accs_v[1, b, s]
                for k in range(2, NACC):
                    acc = acc + accs_v[k, b, s]
                out_v[b, s] = acc

        pltpu.sync_copy(out_v, out_hbm.at[pl.ds(base, BPW)])

    return pool(x, table)


def _mlp_tc(pooled_sum, w1t, b1, w2, b2):
    def body(p_ref, w1_ref, b1_ref, w2_ref, b2_ref, o_ref):
        p = p_ref[...] * (1.0 / L)
        h = jnp.dot(p, w1_ref[...], preferred_element_type=jnp.float32) + b1_ref[...]
        h = jnp.maximum(h, 0.0)
        o_ref[...] = jnp.sum(h * w2_ref[...], axis=1, keepdims=True) + b2_ref[...]

    return pl.pallas_call(
        body,
        out_shape=jax.ShapeDtypeStruct((B, 1), jnp.float32),
    )(pooled_sum, w1t, b1, w2, b2)


def kernel(x, emb, W1, b1, W2, b2):
    table = _relayout_tc(emb.T).reshape(2 * V, D)
    pooled_sum = _pool_sc(x, table)
    out = _mlp_tc(
        pooled_sum,
        W1.T,
        b1.reshape(1, H),
        W2.reshape(1, H),
        b2.reshape(1, 1),
    )
    return out.reshape(B)


# speedup vs baseline: 3.0233x; 1.0173x over previous
import functools

import jax
import jax.numpy as jnp
from jax import lax
from jax.experimental import pallas as pl
from jax.experimental.pallas import tpu as pltpu
from jax.experimental.pallas import tpu_sc as plsc

V = 1000000
B = 4096
L = 200
D = 64
H = 32
NC = 2
NS = 16
NW = NC * NS
BPW = B // NW
NACC = 8
NV = D // 16
TB = 20480


def _relayout_tc(emb_t):
    def body(in_ref, o_ref):
        t = in_ref[...].T
        o_ref[...] = jnp.concatenate([t, t], axis=1)

    return pl.pallas_call(
        body,
        grid=((V + TB - 1) // TB,),
        in_specs=[pl.BlockSpec((D, TB), lambda i: (0, i))],
        out_specs=pl.BlockSpec((TB, 2 * D), lambda i: (i, 0)),
        out_shape=jax.ShapeDtypeStruct((V, 2 * D), jnp.float32),
    )(emb_t)


def _pool_sc(x, table):
    mesh = plsc.VectorSubcoreMesh(core_axis_name="core", subcore_axis_name="subcore")

    @functools.partial(
        pl.kernel,
        out_type=jax.ShapeDtypeStruct((B, D), jnp.float32),
        mesh=mesh,
        scratch_types=[
            pltpu.VMEM((BPW, L), jnp.int32),
            pltpu.VMEM((L, BPW), jnp.int32),
            pltpu.VMEM((NACC, BPW, D), jnp.float32),
            pltpu.VMEM((BPW, D), jnp.float32),
        ]
        + [pltpu.SemaphoreType.DMA] * NACC,
        compiler_params=pltpu.CompilerParams(
            use_tc_tiling_on_sc=False, needs_layout_passes=False
        ),
    )
    def pool(x_hbm, table_hbm, out_hbm, idx_raw, idx_v, accs_v, out_v, *sems):
        wid = lax.axis_index("subcore") * NC + lax.axis_index("core")
        base = wid * BPW
        pltpu.sync_copy(x_hbm.at[pl.ds(base, BPW)], idx_raw)

        lanes = lax.iota(jnp.int32, 16)

        @pl.loop(0, L)
        def _(j):
            cols = jnp.zeros((16,), jnp.int32) + j
            for g in range(BPW // 16):
                v = plsc.load_gather(idx_raw, [lanes + 16 * g, cols])
                idx_v[j, pl.ds(16 * g, 16)] = v + v

        for k in range(NACC):
            pltpu.async_copy(table_hbm.at[idx_v.at[k]], accs_v.at[k], sems[k])

        @pl.loop(NACC, L, step=NACC)
        def _(j):
            for k in range(NACC):
                pltpu.make_async_copy(
                    table_hbm.at[idx_v.at[0]], accs_v.at[k], sems[k]
                ).wait()
                pltpu.async_copy(
                    table_hbm.at[idx_v.at[j + k]], accs_v.at[k], sems[k], add=True
                )

        for k in range(NACC):
            pltpu.make_async_copy(
                table_hbm.at[idx_v.at[0]], accs_v.at[k], sems[k]
            ).wait()

        @pl.loop(0, BPW)
        def _(b):
            for i in range(NV):
                s = pl.ds(16 * i, 16)
                acc = accs_v[0, b, s] + accs_v[1, b, s]
                for k in range(2, NACC):
                    acc = acc + accs_v[k, b, s]
                out_v[b, s] = acc

        pltpu.sync_copy(out_v, out_hbm.at[pl.ds(base, BPW)])

    return pool(x, table)


def _mlp_tc(pooled_sum, w1t, b1, w2, b2):
    def body(p_ref, w1_ref, b1_ref, w2_ref, b2_ref, o_ref):
        p = p_ref[...] * (1.0 / L)
        h = jnp.dot(p, w1_ref[...], preferred_element_type=jnp.float32) + b1_ref[...]
        h = jnp.maximum(h, 0.0)
        o_ref[...] = jnp.sum(h * w2_ref[...], axis=1, keepdims=True) + b2_ref[...]

    return pl.pallas_call(
        body,
        out_shape=jax.ShapeDtypeStruct((B, 1), jnp.float32),
    )(pooled_sum, w1t, b1, w2, b2)


def kernel(x, emb, W1, b1, W2, b2):
    table = _relayout_tc(emb.T).reshape(2 * V, D)
    pooled_sum = _pool_sc(x, table)
    out = _mlp_tc(
        pooled_sum,
        W1.T,
        b1.reshape(1, H),
        W2.reshape(1, H),
        b2.reshape(1, 1),
    )
    return out.reshape(B)
